# baseline (speedup 1.0000x reference)
import functools

import jax
import jax.numpy as jnp
from jax import lax
from jax.experimental import pallas as pl
from jax.experimental.pallas import tpu as pltpu
from jax.experimental.pallas import tpu_sc as plsc

N = 10000
E = 320000
D = 128
C = 16
NC = 2
NS = 16
NW = NC * NS
K = 128
G = 8
NB = 10
CH = NB * G
CHB = (NB + 1) * G
EW = CH * K
EP = EW * NW
EW1 = -(-E // (NW * K)) * K
EP1 = EW1 * NW
CHA = 80
EWT = (CHA + 2) * K
EPT = EWT * NW
NP = 10016
RPT = NP // NS

_mesh = plsc.VectorSubcoreMesh(
    core_axis_name="c", subcore_axis_name="s", num_cores=NC, num_subcores=NS)


@functools.partial(
    pl.kernel,
    out_type=jax.ShapeDtypeStruct((NW, NP), jnp.float32),
    mesh=_mesh,
    scratch_types=[
        pltpu.VMEM((CHB, K), jnp.int32),
        pltpu.VMEM((NP,), jnp.float32),
    ],
    compiler_params=pltpu.CompilerParams(needs_layout_passes=False),
)
def _deg_kernel(dst_hbm, degp_hbm, dstl, degl):
    cid = lax.axis_index("c")
    sid = lax.axis_index("s")
    wid = cid * NS + sid
    zeros16 = jnp.zeros((16,), jnp.float32)
    ones16 = jnp.ones((16,), jnp.float32)

    pltpu.sync_copy(dst_hbm.at[wid], dstl)

    def zbody(i, carry):
        degl[pl.ds(i * 16, 16)] = zeros16
        return carry
    lax.fori_loop(0, NP // 16, zbody, 0)

    def chunk(ci, carry):
        for j in range(K // 16):
            idx = dstl[ci, pl.ds(j * 16, 16)]
            plsc.addupdate_scatter(degl, [idx], ones16)
        return carry
    lax.fori_loop(0, CH, chunk, 0)

    pltpu.sync_copy(degl, degp_hbm.at[wid])


def _make_agg_ser(width):
    @functools.partial(
        pl.kernel,
        out_type=jax.ShapeDtypeStruct((NC, NP, width), jnp.float32),
        mesh=_mesh,
        scratch_types=[
            pltpu.VMEM((K,), jnp.int32),
            pltpu.VMEM((K,), jnp.int32),
            pltpu.VMEM((K, width), jnp.float32),
            pltpu.VMEM_SHARED((NP, width), jnp.float32),
            pltpu.SemaphoreType.DMA,
        ],
        compiler_params=pltpu.CompilerParams(use_tc_tiling_on_sc=False),
    )
    def _agg(g_hbm, src_hbm, dst_hbm, zeros_hbm, out_hbm,
             srcv, dstv, rows, acc, sem):
        cid = lax.axis_index("c")
        sid = lax.axis_index("s")
        wid = cid * NS + sid
        r0 = sid * RPT
        pltpu.sync_copy(zeros_hbm.at[pl.ds(r0, RPT)], acc.at[pl.ds(r0, RPT)])
        plsc.subcore_barrier()

        base = wid * EWT

        def chunk(ci, carry):
            off = base + ci * K
            pltpu.sync_copy(src_hbm.at[pl.ds(off, K)], srcv)
            pltpu.sync_copy(dst_hbm.at[pl.ds(off, K)], dstv)
            pltpu.async_copy(g_hbm.at[srcv], rows, sem).wait()
            pltpu.sync_copy(rows, acc.at[dstv], add=True)
            return carry
        lax.fori_loop(0, CHA, chunk, 0)

        plsc.subcore_barrier()
        pltpu.sync_copy(acc.at[pl.ds(r0, RPT)],
                        out_hbm.at[cid, pl.ds(r0, RPT)])
    return _agg


def _make_agg_pipe(width):
    @functools.partial(
        pl.kernel,
        out_type=jax.ShapeDtypeStruct((NC, NP, width), jnp.float32),
        mesh=_mesh,
        scratch_types=[
            pltpu.VMEM((K,), jnp.int32),
            pltpu.VMEM((K,), jnp.int32),
            pltpu.VMEM((K,), jnp.int32),
            pltpu.VMEM((K,), jnp.int32),
            pltpu.VMEM((K, width), jnp.float32),
            pltpu.VMEM((K, width), jnp.float32),
            pltpu.VMEM_SHARED((NP, width), jnp.float32),
            pltpu.SemaphoreType.DMA,
            pltpu.SemaphoreType.DMA,
            pltpu.SemaphoreType.DMA,
            pltpu.SemaphoreType.DMA,
        ],
        compiler_params=pltpu.CompilerParams(use_tc_tiling_on_sc=False),
    )
    def _agg(g_hbm, src_hbm, dst_hbm, zeros_hbm, out_hbm,
             srcv0, srcv1, dstv0, dstv1, rows0, rows1,
             acc, isem0, isem1, gsem0, gsem1):
        cid = lax.axis_index("c")
        sid = lax.axis_index("s")
        wid = cid * NS + sid
        r0 = sid * RPT
        srcv = (srcv0, srcv1)
        dstv = (dstv0, dstv1)
        rows = (rows0, rows1)
        isem = (isem0, isem1)
        gsem = (gsem0, gsem1)
        base = wid * EWT

        def idx_load(c, p):
            pltpu.async_copy(src_hbm.at[pl.ds(base + c * K, K)], srcv[p],
                             isem[p])
            pltpu.async_copy(dst_hbm.at[pl.ds(base + c * K, K)], dstv[p],
                             isem[p])

        def idx_wait(c, p):
            pltpu.make_async_copy(src_hbm.at[pl.ds(base + c * K, K)], srcv[p],
                                  isem[p]).wait()
            pltpu.make_async_copy(dst_hbm.at[pl.ds(base + c * K, K)], dstv[p],
                                  isem[p]).wait()

        pltpu.sync_copy(zeros_hbm.at[pl.ds(r0, RPT)], acc.at[pl.ds(r0, RPT)])
        idx_load(0, 0)
        idx_load(1, 1)
        plsc.subcore_barrier()
        idx_wait(0, 0)
        pltpu.async_copy(g_hbm.at[srcv0], rows0, gsem0)

        def body(i, carry):
            a = 2 * i
            idx_wait(a + 1, 1)
            pltpu.async_copy(g_hbm.at[srcv[1]], rows[1], gsem[1])
            pltpu.make_async_copy(g_hbm.at[srcv[0]], rows[0], gsem[0]).wait()
            pltpu.sync_copy(rows[0], acc.at[dstv[0]], add=True)
            idx_load(a + 2, 0)
            pltpu.make_async_copy(g_hbm.at[srcv[1]], rows[1], gsem[1]).wait()
            pltpu.sync_copy(rows[1], acc.at[dstv[1]], add=True)
            idx_load(a + 3, 1)
            idx_wait(a + 2, 0)
            pltpu.async_copy(g_hbm.at[srcv[0]], rows[0], gsem[0])
            return carry
        lax.fori_loop(0, CHA // 2, body, 0)

        pltpu.make_async_copy(g_hbm.at[srcv0], rows0, gsem0).wait()
        idx_wait(CHA + 1, 1)

        plsc.subcore_barrier()
        pltpu.sync_copy(acc.at[pl.ds(r0, RPT)],
                        out_hbm.at[cid, pl.ds(r0, RPT)])
    return _agg


_agg_d = _make_agg_ser(D)
_agg_c = _make_agg_pipe(C)


def _mm_body(x_ref, w_ref, o_ref):
    o_ref[...] = jnp.dot(x_ref[...], w_ref[...],
                         preferred_element_type=jnp.float32)


def _norm_scale_body(degp_ref, h_ref, g_ref, dis_ref):
    deg = jnp.sum(degp_ref[...], axis=0)[:N] + 1.0
    dis = lax.rsqrt(deg)
    dis_ref[...] = dis
    g_ref[...] = h_ref[...] * dis[:, None]


def _mid_body(sp_ref, g1_ref, dis_ref, b1_ref, w2_ref, g2_ref):
    s = sp_ref[0, :N, :] + sp_ref[1, :N, :]
    dis = dis_ref[...]
    out1 = jnp.maximum((s + g1_ref[...]) * dis[:, None] + b1_ref[...][None, :],
                       0.0)
    g2_ref[...] = jnp.dot(out1, w2_ref[...],
                          preferred_element_type=jnp.float32) * dis[:, None]


def _fin_body(sp_ref, g2_ref, dis_ref, b2_ref, t_ref, pred_ref, prob_ref):
    s = sp_ref[0, :N, :] + sp_ref[1, :N, :]
    logits = (s + g2_ref[...]) * dis_ref[...][:, None] + b2_ref[...][None, :]
    probs = jax.nn.sigmoid(logits)
    prob_ref[...] = probs
    pred_ref[...] = (probs > t_ref[0]).astype(jnp.float32)


@jax.jit
def kernel(x, edge_index, W1, b1, W2, b2, threshold):
    f32 = jnp.float32
    npad = EP - E
    srcp = jnp.concatenate([edge_index[0], jnp.zeros((npad,), jnp.int32)])
    dstp = jnp.concatenate([edge_index[1], jnp.full((npad,), N, jnp.int32)])
    dst3 = jnp.concatenate(
        [dstp.reshape(NW, CH, K),
         jnp.full((NW, G, K), N, jnp.int32)], axis=1)
    srcw = jnp.concatenate(
        [edge_index[0], jnp.zeros((NW * CHA * K - E,), jnp.int32)]
    ).reshape(NW, CHA * K)
    dstw = jnp.concatenate(
        [edge_index[1], jnp.full((NW * CHA * K - E,), N, jnp.int32)]
    ).reshape(NW, CHA * K)
    srcp1 = jnp.concatenate(
        [srcw, jnp.zeros((NW, 2 * K), jnp.int32)], axis=1).reshape(EPT)
    dstp1 = jnp.concatenate(
        [dstw, jnp.full((NW, 2 * K), N, jnp.int32)], axis=1).reshape(EPT)
    zeros_d = jnp.zeros((NP, D), f32)
    zeros_c = jnp.zeros((NP, C), f32)
    thr = jnp.reshape(threshold.astype(f32), (1,))

    degp = _deg_kernel(dst3)
    h1 = pl.pallas_call(
        _mm_body, out_shape=jax.ShapeDtypeStruct((N, D), f32))(x, W1)
    g1, dis = pl.pallas_call(
        _norm_scale_body,
        out_shape=(jax.ShapeDtypeStruct((N, D), f32),
                   jax.ShapeDtypeStruct((N,), f32)))(degp, h1)
    s1p = _agg_d(g1, srcp1, dstp1, zeros_d)
    g2 = pl.pallas_call(
        _mid_body, out_shape=jax.ShapeDtypeStruct((N, C), f32))(
            s1p, g1, dis, b1, W2)
    s2p = _agg_c(g2, srcp1, dstp1, zeros_c)
    preds, probs = pl.pallas_call(
        _fin_body,
        out_shape=(jax.ShapeDtypeStruct((N, C), f32),
                   jax.ShapeDtypeStruct((N, C), f32)))(
            s2p, g2, dis, b2, thr)
    return (preds, probs)


# speedup vs baseline: 1.1303x; 1.1303x over previous
import functools

import jax
import jax.numpy as jnp
from jax import lax
from jax.experimental import pallas as pl
from jax.experimental.pallas import tpu as pltpu
from jax.experimental.pallas import tpu_sc as plsc

N = 10000
E = 320000
D = 128
C = 16
NC = 2
NS = 16
NW = NC * NS
K = 128
G = 8
NB = 10
CH = NB * G
CHB = (NB + 1) * G
EW = CH * K
EP = EW * NW
EW1 = -(-E // (NW * K)) * K
EP1 = EW1 * NW
NCH0 = 61
NCH1 = 97
NP = 10016
RPT = NP // NS

_mesh = plsc.VectorSubcoreMesh(
    core_axis_name="c", subcore_axis_name="s", num_cores=NC, num_subcores=NS)


@functools.partial(
    pl.kernel,
    out_type=jax.ShapeDtypeStruct((NW, NP), jnp.float32),
    mesh=_mesh,
    scratch_types=[
        pltpu.VMEM((CHB, K), jnp.int32),
        pltpu.VMEM((NP,), jnp.float32),
    ],
    compiler_params=pltpu.CompilerParams(needs_layout_passes=False),
)
def _deg_kernel(dst_hbm, degp_hbm, dstl, degl):
    cid = lax.axis_index("c")
    sid = lax.axis_index("s")
    wid = cid * NS + sid
    zeros16 = jnp.zeros((16,), jnp.float32)
    ones16 = jnp.ones((16,), jnp.float32)

    pltpu.sync_copy(dst_hbm.at[wid], dstl)

    def zbody(i, carry):
        degl[pl.ds(i * 16, 16)] = zeros16
        return carry
    lax.fori_loop(0, NP // 16, zbody, 0)

    def chunk(ci, carry):
        for j in range(K // 16):
            idx = dstl[ci, pl.ds(j * 16, 16)]
            plsc.addupdate_scatter(degl, [idx], ones16)
        return carry
    lax.fori_loop(0, CH, chunk, 0)

    pltpu.sync_copy(degl, degp_hbm.at[wid])


def _make_agg(width, nch0, nch1):
    @functools.partial(
        pl.kernel,
        out_type=jax.ShapeDtypeStruct((NC, NP, width), jnp.float32),
        mesh=_mesh,
        scratch_types=[
            pltpu.VMEM((K,), jnp.int32),
            pltpu.VMEM((K,), jnp.int32),
            pltpu.VMEM((K, width), jnp.float32),
            pltpu.VMEM_SHARED((NP, width), jnp.float32),
            pltpu.SemaphoreType.DMA,
        ],
        compiler_params=pltpu.CompilerParams(use_tc_tiling_on_sc=False),
    )
    def _agg(g_hbm, src_hbm, dst_hbm, zeros_hbm, out_hbm,
             srcv, dstv, rows, acc, sem):
        cid = lax.axis_index("c")
        sid = lax.axis_index("s")
        wid = cid * NS + sid
        r0 = sid * RPT
        pltpu.sync_copy(zeros_hbm.at[pl.ds(r0, RPT)], acc.at[pl.ds(r0, RPT)])
        plsc.subcore_barrier()

        ncha = jnp.where(cid == 0, nch0, nch1)
        cbase = jnp.where(cid == 0, sid * nch0, NS * nch0 + sid * nch1)

        def chunk(ci, carry):
            off = (cbase + ci) * K
            pltpu.sync_copy(src_hbm.at[pl.ds(off, K)], srcv)
            pltpu.sync_copy(dst_hbm.at[pl.ds(off, K)], dstv)
            pltpu.async_copy(g_hbm.at[srcv], rows, sem).wait()
            pltpu.sync_copy(rows, acc.at[dstv], add=True)
            return carry
        lax.fori_loop(0, ncha, chunk, 0)

        plsc.subcore_barrier()
        pltpu.sync_copy(acc.at[pl.ds(r0, RPT)],
                        out_hbm.at[cid, pl.ds(r0, RPT)])
    return _agg


_agg_d = _make_agg(D, NCH0, NCH1)
_agg_c = _make_agg(C, 79, 79)


def _mm_body(x_ref, w_ref, o_ref):
    o_ref[...] = jnp.dot(x_ref[...], w_ref[...],
                         preferred_element_type=jnp.float32)


def _norm_scale_body(degp_ref, h_ref, g_ref, dis_ref):
    deg = jnp.sum(degp_ref[...], axis=0)[:N] + 1.0
    dis = lax.rsqrt(deg)
    dis_ref[...] = dis
    g_ref[...] = h_ref[...] * dis[:, None]


def _mid_body(sp_ref, g1_ref, dis_ref, b1_ref, w2_ref, g2_ref):
    s = sp_ref[0, :N, :] + sp_ref[1, :N, :]
    dis = dis_ref[...]
    out1 = jnp.maximum((s + g1_ref[...]) * dis[:, None] + b1_ref[...][None, :],
                       0.0)
    g2_ref[...] = jnp.dot(out1, w2_ref[...],
                          preferred_element_type=jnp.float32) * dis[:, None]


def _fin_body(sp_ref, g2_ref, dis_ref, b2_ref, t_ref, pred_ref, prob_ref):
    s = sp_ref[0, :N, :] + sp_ref[1, :N, :]
    logits = (s + g2_ref[...]) * dis_ref[...][:, None] + b2_ref[...][None, :]
    probs = jax.nn.sigmoid(logits)
    prob_ref[...] = probs
    pred_ref[...] = (probs > t_ref[0]).astype(jnp.float32)


@jax.jit
def kernel(x, edge_index, W1, b1, W2, b2, threshold):
    f32 = jnp.float32
    npad = EP - E
    srcp = jnp.concatenate([edge_index[0], jnp.zeros((npad,), jnp.int32)])
    dstp = jnp.concatenate([edge_index[1], jnp.full((npad,), N, jnp.int32)])
    dst3 = jnp.concatenate(
        [dstp.reshape(NW, CH, K),
         jnp.full((NW, G, K), N, jnp.int32)], axis=1)
    npad1 = EP1 - E
    srcp1 = jnp.concatenate([edge_index[0], jnp.zeros((npad1,), jnp.int32)])
    dstp1 = jnp.concatenate([edge_index[1], jnp.full((npad1,), N, jnp.int32)])
    zeros_d = jnp.zeros((NP, D), f32)
    zeros_c = jnp.zeros((NP, C), f32)
    thr = jnp.reshape(threshold.astype(f32), (1,))

    degp = _deg_kernel(dst3)
    h1 = pl.pallas_call(
        _mm_body, out_shape=jax.ShapeDtypeStruct((N, D), f32))(x, W1)
    g1, dis = pl.pallas_call(
        _norm_scale_body,
        out_shape=(jax.ShapeDtypeStruct((N, D), f32),
                   jax.ShapeDtypeStruct((N,), f32)))(degp, h1)
    s1p = _agg_d(g1, srcp1, dstp1, zeros_d)
    g2 = pl.pallas_call(
        _mid_body, out_shape=jax.ShapeDtypeStruct((N, C), f32))(
            s1p, g1, dis, b1, W2)
    s2p = _agg_c(g2, srcp1, dstp1, zeros_c)
    preds, probs = pl.pallas_call(
        _fin_body,
        out_shape=(jax.ShapeDtypeStruct((N, C), f32),
                   jax.ShapeDtypeStruct((N, C), f32)))(
            s2p, g2, dis, b2, thr)
    return (preds, probs)


# speedup vs baseline: 1.3165x; 1.1647x over previous
import functools

import jax
import jax.numpy as jnp
from jax import lax
from jax.experimental import pallas as pl
from jax.experimental.pallas import tpu as pltpu
from jax.experimental.pallas import tpu_sc as plsc

N = 10000
E = 320000
D = 128
C = 16
NC = 2
NS = 16
NW = NC * NS
K = 128
G = 8
NB = 10
CH = NB * G
CHB = (NB + 1) * G
EW = CH * K
EP = EW * NW
EW1 = -(-E // (NW * K)) * K
EP1 = EW1 * NW
NCH0 = 97
NCH1 = 61
NP = 10016
RPT = NP // NS

_mesh = plsc.VectorSubcoreMesh(
    core_axis_name="c", subcore_axis_name="s", num_cores=NC, num_subcores=NS)


@functools.partial(
    pl.kernel,
    out_type=jax.ShapeDtypeStruct((NW, NP), jnp.float32),
    mesh=_mesh,
    scratch_types=[
        pltpu.VMEM((CHB, K), jnp.int32),
        pltpu.VMEM((NP,), jnp.float32),
    ],
    compiler_params=pltpu.CompilerParams(needs_layout_passes=False),
)
def _deg_kernel(dst_hbm, degp_hbm, dstl, degl):
    cid = lax.axis_index("c")
    sid = lax.axis_index("s")
    wid = cid * NS + sid
    zeros16 = jnp.zeros((16,), jnp.float32)
    ones16 = jnp.ones((16,), jnp.float32)

    pltpu.sync_copy(dst_hbm.at[wid], dstl)

    def zbody(i, carry):
        degl[pl.ds(i * 16, 16)] = zeros16
        return carry
    lax.fori_loop(0, NP // 16, zbody, 0)

    def chunk(ci, carry):
        for j in range(K // 16):
            idx = dstl[ci, pl.ds(j * 16, 16)]
            plsc.addupdate_scatter(degl, [idx], ones16)
        return carry
    lax.fori_loop(0, CH, chunk, 0)

    pltpu.sync_copy(degl, degp_hbm.at[wid])


def _make_agg(width, nch0, nch1):
    @functools.partial(
        pl.kernel,
        out_type=jax.ShapeDtypeStruct((NC, NP, width), jnp.float32),
        mesh=_mesh,
        scratch_types=[
            pltpu.VMEM((K,), jnp.int32),
            pltpu.VMEM((K,), jnp.int32),
            pltpu.VMEM((K, width), jnp.float32),
            pltpu.VMEM_SHARED((NP, width), jnp.float32),
            pltpu.SemaphoreType.DMA,
        ],
        compiler_params=pltpu.CompilerParams(use_tc_tiling_on_sc=False),
    )
    def _agg(g_hbm, src_hbm, dst_hbm, zeros_hbm, out_hbm,
             srcv, dstv, rows, acc, sem):
        cid = lax.axis_index("c")
        sid = lax.axis_index("s")
        wid = cid * NS + sid
        r0 = sid * RPT
        pltpu.sync_copy(zeros_hbm.at[pl.ds(r0, RPT)], acc.at[pl.ds(r0, RPT)])
        plsc.subcore_barrier()

        ncha = jnp.where(cid == 0, nch0, nch1)
        cbase = jnp.where(cid == 0, sid * nch0, NS * nch0 + sid * nch1)

        def chunk(ci, carry):
            off = (cbase + ci) * K
            pltpu.sync_copy(src_hbm.at[pl.ds(off, K)], srcv)
            pltpu.sync_copy(dst_hbm.at[pl.ds(off, K)], dstv)
            pltpu.async_copy(g_hbm.at[srcv], rows, sem).wait()
            pltpu.sync_copy(rows, acc.at[dstv], add=True)
            return carry
        lax.fori_loop(0, ncha, chunk, 0)

        plsc.subcore_barrier()
        pltpu.sync_copy(acc.at[pl.ds(r0, RPT)],
                        out_hbm.at[cid, pl.ds(r0, RPT)])
    return _agg


_agg_d = _make_agg(D, NCH0, NCH1)
_agg_c = _make_agg(C, 79, 79)


def _mm_body(x_ref, w_ref, o_ref):
    o_ref[...] = jnp.dot(x_ref[...], w_ref[...],
                         preferred_element_type=jnp.float32)


def _norm_scale_body(degp_ref, h_ref, g_ref, dis_ref):
    deg = jnp.sum(degp_ref[...], axis=0)[:N] + 1.0
    dis = lax.rsqrt(deg)
    dis_ref[...] = dis
    g_ref[...] = h_ref[...] * dis[:, None]


def _mid_body(sp_ref, g1_ref, dis_ref, b1_ref, w2_ref, g2_ref):
    s = sp_ref[0, :N, :] + sp_ref[1, :N, :]
    dis = dis_ref[...]
    out1 = jnp.maximum((s + g1_ref[...]) * dis[:, None] + b1_ref[...][None, :],
                       0.0)
    g2_ref[...] = jnp.dot(out1, w2_ref[...],
                          preferred_element_type=jnp.float32) * dis[:, None]


def _fin_body(sp_ref, g2_ref, dis_ref, b2_ref, t_ref, pred_ref, prob_ref):
    s = sp_ref[0, :N, :] + sp_ref[1, :N, :]
    logits = (s + g2_ref[...]) * dis_ref[...][:, None] + b2_ref[...][None, :]
    probs = jax.nn.sigmoid(logits)
    prob_ref[...] = probs
    pred_ref[...] = (probs > t_ref[0]).astype(jnp.float32)


@jax.jit
def kernel(x, edge_index, W1, b1, W2, b2, threshold):
    f32 = jnp.float32
    npad = EP - E
    srcp = jnp.concatenate([edge_index[0], jnp.zeros((npad,), jnp.int32)])
    dstp = jnp.concatenate([edge_index[1], jnp.full((npad,), N, jnp.int32)])
    dst3 = jnp.concatenate(
        [dstp.reshape(NW, CH, K),
         jnp.full((NW, G, K), N, jnp.int32)], axis=1)
    npad1 = EP1 - E
    srcp1 = jnp.concatenate([edge_index[0], jnp.zeros((npad1,), jnp.int32)])
    dstp1 = jnp.concatenate([edge_index[1], jnp.full((npad1,), N, jnp.int32)])
    zeros_d = jnp.zeros((NP, D), f32)
    zeros_c = jnp.zeros((NP, C), f32)
    thr = jnp.reshape(threshold.astype(f32), (1,))

    degp = _deg_kernel(dst3)
    h1 = pl.pallas_call(
        _mm_body, out_shape=jax.ShapeDtypeStruct((N, D), f32))(x, W1)
    g1, dis = pl.pallas_call(
        _norm_scale_body,
        out_shape=(jax.ShapeDtypeStruct((N, D), f32),
                   jax.ShapeDtypeStruct((N,), f32)))(degp, h1)
    s1p = _agg_d(g1, srcp1, dstp1, zeros_d)
    g2 = pl.pallas_call(
        _mid_body, out_shape=jax.ShapeDtypeStruct((N, C), f32))(
            s1p, g1, dis, b1, W2)
    s2p = _agg_c(g2, srcp1, dstp1, zeros_c)
    preds, probs = pl.pallas_call(
        _fin_body,
        out_shape=(jax.ShapeDtypeStruct((N, C), f32),
                   jax.ShapeDtypeStruct((N, C), f32)))(
            s2p, g2, dis, b2, thr)
    return (preds, probs)


# speedup vs baseline: 1.5903x; 1.2080x over previous
import functools

import jax
import jax.numpy as jnp
from jax import lax
from jax.experimental import pallas as pl
from jax.experimental.pallas import tpu as pltpu
from jax.experimental.pallas import tpu_sc as plsc

N = 10000
E = 320000
D = 128
C = 16
NC = 2
NS = 16
NW = NC * NS
K = 128
G = 8
NB = 10
CH = NB * G
CHB = (NB + 1) * G
EW = CH * K
EP = EW * NW
EW1 = -(-E // (NW * K)) * K
EP1 = EW1 * NW
NCH0 = 100
NCH1 = 58
PCH0 = 94
PCH1 = 64
NP = 10016
RPT = NP // NS

_mesh = plsc.VectorSubcoreMesh(
    core_axis_name="c", subcore_axis_name="s", num_cores=NC, num_subcores=NS)


@functools.partial(
    pl.kernel,
    out_type=jax.ShapeDtypeStruct((NW, NP), jnp.float32),
    mesh=_mesh,
    scratch_types=[
        pltpu.VMEM((CHB, K), jnp.int32),
        pltpu.VMEM((NP,), jnp.float32),
    ],
    compiler_params=pltpu.CompilerParams(needs_layout_passes=False),
)
def _deg_kernel(dst_hbm, degp_hbm, dstl, degl):
    cid = lax.axis_index("c")
    sid = lax.axis_index("s")
    wid = cid * NS + sid
    zeros16 = jnp.zeros((16,), jnp.float32)
    ones16 = jnp.ones((16,), jnp.float32)

    pltpu.sync_copy(dst_hbm.at[wid], dstl)

    def zbody(i, carry):
        degl[pl.ds(i * 16, 16)] = zeros16
        return carry
    lax.fori_loop(0, NP // 16, zbody, 0)

    def chunk(ci, carry):
        for j in range(K // 16):
            idx = dstl[ci, pl.ds(j * 16, 16)]
            plsc.addupdate_scatter(degl, [idx], ones16)
        return carry
    lax.fori_loop(0, CH, chunk, 0)

    pltpu.sync_copy(degl, degp_hbm.at[wid])


def _make_agg(width, nch0, nch1):
    @functools.partial(
        pl.kernel,
        out_type=jax.ShapeDtypeStruct((NC, NP, width), jnp.float32),
        mesh=_mesh,
        scratch_types=[
            pltpu.VMEM((K,), jnp.int32),
            pltpu.VMEM((K,), jnp.int32),
            pltpu.VMEM((K, width), jnp.float32),
            pltpu.VMEM_SHARED((NP, width), jnp.float32),
            pltpu.SemaphoreType.DMA,
        ],
        compiler_params=pltpu.CompilerParams(use_tc_tiling_on_sc=False),
    )
    def _agg(g_hbm, src_hbm, dst_hbm, zeros_hbm, out_hbm,
             srcv, dstv, rows, acc, sem):
        cid = lax.axis_index("c")
        sid = lax.axis_index("s")
        wid = cid * NS + sid
        r0 = sid * RPT
        pltpu.sync_copy(zeros_hbm.at[pl.ds(r0, RPT)], acc.at[pl.ds(r0, RPT)])
        plsc.subcore_barrier()

        ncha = jnp.where(cid == 0, nch0, nch1)
        cbase = jnp.where(cid == 0, sid * nch0, NS * nch0 + sid * nch1)

        def chunk(ci, carry):
            off = (cbase + ci) * K
            pltpu.sync_copy(src_hbm.at[pl.ds(off, K)], srcv)
            pltpu.sync_copy(dst_hbm.at[pl.ds(off, K)], dstv)
            pltpu.async_copy(g_hbm.at[srcv], rows, sem).wait()
            pltpu.sync_copy(rows, acc.at[dstv], add=True)
            return carry
        lax.fori_loop(0, ncha, chunk, 0)

        plsc.subcore_barrier()
        pltpu.sync_copy(acc.at[pl.ds(r0, RPT)],
                        out_hbm.at[cid, pl.ds(r0, RPT)])
    return _agg


def _make_agg_pipe(width, nch0, nch1):
    @functools.partial(
        pl.kernel,
        out_type=jax.ShapeDtypeStruct((NC, NP, width), jnp.float32),
        mesh=_mesh,
        scratch_types=[
            pltpu.VMEM((K,), jnp.int32),
            pltpu.VMEM((K,), jnp.int32),
            pltpu.VMEM((K,), jnp.int32),
            pltpu.VMEM((K,), jnp.int32),
            pltpu.VMEM((K, width), jnp.float32),
            pltpu.VMEM((K, width), jnp.float32),
            pltpu.VMEM_SHARED((NP, width), jnp.float32),
            pltpu.SemaphoreType.DMA,
            pltpu.SemaphoreType.DMA,
            pltpu.SemaphoreType.DMA,
            pltpu.SemaphoreType.DMA,
        ],
        compiler_params=pltpu.CompilerParams(use_tc_tiling_on_sc=False),
    )
    def _agg(g_hbm, src_hbm, dst_hbm, zeros_hbm, out_hbm,
             srcv0, srcv1, dstv0, dstv1, rows0, rows1,
             acc, isem0, isem1, gsem0, gsem1):
        cid = lax.axis_index("c")
        sid = lax.axis_index("s")
        r0 = sid * RPT
        srcv = (srcv0, srcv1)
        dstv = (dstv0, dstv1)
        rows = (rows0, rows1)
        isem = (isem0, isem1)
        gsem = (gsem0, gsem1)
        ncha = jnp.where(cid == 0, nch0, nch1)
        cbase = jnp.where(cid == 0, sid * nch0, NS * nch0 + sid * nch1)

        def idx_load(c, p):
            off = (cbase + c) * K
            pltpu.async_copy(src_hbm.at[pl.ds(off, K)], srcv[p], isem[p])
            pltpu.async_copy(dst_hbm.at[pl.ds(off, K)], dstv[p], isem[p])

        def idx_wait(c, p):
            off = (cbase + c) * K
            pltpu.make_async_copy(src_hbm.at[pl.ds(off, K)], srcv[p],
                                  isem[p]).wait()
            pltpu.make_async_copy(dst_hbm.at[pl.ds(off, K)], dstv[p],
                                  isem[p]).wait()

        pltpu.sync_copy(zeros_hbm.at[pl.ds(r0, RPT)], acc.at[pl.ds(r0, RPT)])
        idx_load(0, 0)
        idx_load(1, 1)
        plsc.subcore_barrier()
        idx_wait(0, 0)
        pltpu.async_copy(g_hbm.at[srcv0], rows0, gsem0)

        def body(i, carry):
            a = 2 * i
            idx_wait(a + 1, 1)
            pltpu.async_copy(g_hbm.at[srcv[1]], rows[1], gsem[1])
            pltpu.make_async_copy(g_hbm.at[srcv[0]], rows[0], gsem[0]).wait()
            pltpu.sync_copy(rows[0], acc.at[dstv[0]], add=True)
            idx_load(a + 2, 0)
            pltpu.make_async_copy(g_hbm.at[srcv[1]], rows[1], gsem[1]).wait()
            pltpu.sync_copy(rows[1], acc.at[dstv[1]], add=True)
            idx_load(a + 3, 1)
            idx_wait(a + 2, 0)
            pltpu.async_copy(g_hbm.at[srcv[0]], rows[0], gsem[0])
            return carry
        lax.fori_loop(0, ncha // 2, body, 0)

        pltpu.make_async_copy(g_hbm.at[srcv0], rows0, gsem0).wait()
        idx_wait(ncha + 1, 1)

        plsc.subcore_barrier()
        pltpu.sync_copy(acc.at[pl.ds(r0, RPT)],
                        out_hbm.at[cid, pl.ds(r0, RPT)])
    return _agg


_agg_d = _make_agg(D, NCH0, NCH1)
_agg_c = _make_agg_pipe(C, PCH0, PCH1)


def _mm_body(x_ref, w_ref, o_ref):
    o_ref[...] = jnp.dot(x_ref[...], w_ref[...],
                         preferred_element_type=jnp.float32)


def _norm_scale_body(degp_ref, h_ref, g_ref, dis_ref):
    deg = jnp.sum(degp_ref[...], axis=0)[:N] + 1.0
    dis = lax.rsqrt(deg)
    dis_ref[...] = dis
    g_ref[...] = h_ref[...] * dis[:, None]


def _mid_body(sp_ref, g1_ref, dis_ref, b1_ref, w2_ref, g2_ref):
    s = sp_ref[0, :N, :] + sp_ref[1, :N, :]
    dis = dis_ref[...]
    out1 = jnp.maximum((s + g1_ref[...]) * dis[:, None] + b1_ref[...][None, :],
                       0.0)
    g2_ref[...] = jnp.dot(out1, w2_ref[...],
                          preferred_element_type=jnp.float32) * dis[:, None]


def _fin_body(sp_ref, g2_ref, dis_ref, b2_ref, t_ref, pred_ref, prob_ref):
    s = sp_ref[0, :N, :] + sp_ref[1, :N, :]
    logits = (s + g2_ref[...]) * dis_ref[...][:, None] + b2_ref[...][None, :]
    probs = jax.nn.sigmoid(logits)
    prob_ref[...] = probs
    pred_ref[...] = (probs > t_ref[0]).astype(jnp.float32)


@jax.jit
def kernel(x, edge_index, W1, b1, W2, b2, threshold):
    f32 = jnp.float32
    npad = EP - E
    srcp = jnp.concatenate([edge_index[0], jnp.zeros((npad,), jnp.int32)])
    dstp = jnp.concatenate([edge_index[1], jnp.full((npad,), N, jnp.int32)])
    dst3 = jnp.concatenate(
        [dstp.reshape(NW, CH, K),
         jnp.full((NW, G, K), N, jnp.int32)], axis=1)
    npad1 = EP1 + 2 * K - E
    srcp1 = jnp.concatenate([edge_index[0], jnp.zeros((npad1,), jnp.int32)])
    dstp1 = jnp.concatenate([edge_index[1], jnp.full((npad1,), N, jnp.int32)])
    zeros_d = jnp.zeros((NP, D), f32)
    zeros_c = jnp.zeros((NP, C), f32)
    thr = jnp.reshape(threshold.astype(f32), (1,))

    degp = _deg_kernel(dst3)
    h1 = pl.pallas_call(
        _mm_body, out_shape=jax.ShapeDtypeStruct((N, D), f32))(x, W1)
    g1, dis = pl.pallas_call(
        _norm_scale_body,
        out_shape=(jax.ShapeDtypeStruct((N, D), f32),
                   jax.ShapeDtypeStruct((N,), f32)))(degp, h1)
    s1p = _agg_d(g1, srcp1, dstp1, zeros_d)
    g2 = pl.pallas_call(
        _mid_body, out_shape=jax.ShapeDtypeStruct((N, C), f32))(
            s1p, g1, dis, b1, W2)
    s2p = _agg_c(g2, srcp1, dstp1, zeros_c)
    preds, probs = pl.pallas_call(
        _fin_body,
        out_shape=(jax.ShapeDtypeStruct((N, C), f32),
                   jax.ShapeDtypeStruct((N, C), f32)))(
            s2p, g2, dis, b2, thr)
    return (preds, probs)
